# baseline (speedup 1.0000x reference)
import functools

import jax
import jax.numpy as jnp
from jax import lax
from jax.experimental import pallas as pl
from jax.experimental.pallas import tpu as pltpu
from jax.experimental.pallas import tpu_sc as plsc

NUM_ITEM = 1000000
NUM_FACTOR = 64
BATCH = 4096
TARGET_LEN = 20
NEG_LEN = 100
SEQ = TARGET_LEN + NEG_LEN
TOTAL = BATCH * SEQ
HALF_ITEM = NUM_ITEM // 2

NC = 2
NS = 16
NW = NC * NS

_mesh = plsc.VectorSubcoreMesh(
    core_axis_name="c", subcore_axis_name="s", num_cores=NC, num_subcores=NS)


IBLK = 128
NBLK_FULL = NUM_ITEM // IBLK
NBLK_MAIN = 7808
PERW_BLK = NBLK_MAIN // NW


def _transpose_block(in_v, out_v, width):
    jio = jnp.arange(16, dtype=jnp.int32)
    for m in range(width):
        mv = jnp.full((16,), m, dtype=jnp.int32)
        for j0 in range(0, 64, 16):
            vals = plsc.load_gather(in_v, [jio + j0, mv])
            out_v[m // 2, pl.ds((m % 2) * 64 + j0, 16)] = vals


@functools.partial(
    pl.kernel,
    out_type=jax.ShapeDtypeStruct((HALF_ITEM, 128), jnp.float32),
    mesh=_mesh,
    scratch_types=[
        pltpu.VMEM((2, 64, IBLK), jnp.float32),
        pltpu.VMEM((2, IBLK // 2, 128), jnp.float32),
        pltpu.SemaphoreType.DMA((2,)),
        pltpu.SemaphoreType.DMA((2,)),
    ],
    compiler_params=pltpu.CompilerParams(needs_layout_passes=False),
)
def _pack_kernel(wt_hbm, tail_hbm, s1_hbm, in_v, out_v, isem, osem):
    wid = lax.axis_index("s") * NC + lax.axis_index("c")

    def in_dma(k, b):
        t = wid + NW * k
        return pltpu.make_async_copy(
            wt_hbm.at[:, pl.ds(t * IBLK, IBLK)], in_v.at[b], isem.at[b])

    def out_dma(k, b):
        t = wid + NW * k
        return pltpu.make_async_copy(
            out_v.at[b], s1_hbm.at[pl.ds(t * (IBLK // 2), IBLK // 2)],
            osem.at[b])

    in_dma(0, 0).start()
    in_dma(1, 1).start()

    def step(k2, _):
        for b in range(2):
            k = 2 * k2 + b
            in_dma(k, b).wait()
            @pl.when(k2 > 0)
            def _():
                out_dma(k - 2, b).wait()
            _transpose_block(in_v.at[b], out_v.at[b], IBLK)
            out_dma(k, b).start()
            @pl.when(k + 2 < PERW_BLK)
            def _():
                in_dma(k + 2, b).start()
        return _

    lax.fori_loop(0, PERW_BLK // 2, step, None)
    out_dma(PERW_BLK - 2, 0).wait()
    out_dma(PERW_BLK - 1, 1).wait()

    @pl.when(wid < 4)
    def _():
        t = NBLK_MAIN + wid
        pltpu.sync_copy(wt_hbm.at[:, pl.ds(t * IBLK, IBLK)], in_v.at[0])
        _transpose_block(in_v.at[0], out_v.at[0], IBLK)
        pltpu.sync_copy(out_v.at[0],
                        s1_hbm.at[pl.ds(t * (IBLK // 2), IBLK // 2)])

    @pl.when(wid == 4)
    def _():
        pltpu.sync_copy(tail_hbm, out_v.at[1, pl.ds(0, 32)])
        pltpu.sync_copy(out_v.at[1, pl.ds(0, 32)],
                        s1_hbm.at[pl.ds(NBLK_FULL * (IBLK // 2), 32)])


NTB = BATCH // 128
NBUF = 3


@functools.partial(
    pl.kernel,
    out_type=jax.ShapeDtypeStruct((SEQ, 8, NTB, 8, 128), jnp.float32),
    mesh=_mesh,
    scratch_types=[
        pltpu.VMEM((SEQ, 128), jnp.int32),
        pltpu.VMEM((SEQ, 128), jnp.int32),
        pltpu.VMEM((NBUF, 128, 128), jnp.float32),
        pltpu.VMEM((NBUF, 64, 128), jnp.float32),
        pltpu.SemaphoreType.DMA((NBUF,)),
        pltpu.SemaphoreType.DMA((NBUF,)),
    ],
    compiler_params=pltpu.CompilerParams(needs_layout_passes=False),
)
def _gather_kernel(s1_hbm, hidx_hbm, qoff_hbm, out_hbm,
                   hidx_v, qoff_v, gbuf, stage, gsem, ssem):
    wid = lax.axis_index("s") * NC + lax.axis_index("c")

    pltpu.sync_copy(hidx_hbm.at[:, wid], hidx_v)
    pltpu.sync_copy(qoff_hbm.at[:, wid], qoff_v)

    def gather_dma(s, b):
        return pltpu.make_async_copy(
            s1_hbm.at[hidx_v.at[s]], gbuf.at[b], gsem.at[b])

    def store_dma(s, b, tj):
        return pltpu.make_async_copy(
            stage.at[b, pl.ds(tj * 8, 8)], out_hbm.at[s, tj, wid], ssem.at[b])

    for b in range(NBUF):
        gather_dma(b, b).start()

    bio = jnp.arange(16, dtype=jnp.int32)

    def extract(s, b):
        for g in range(8):
            rows = bio + g * 16
            qv = qoff_v[s, pl.ds(g * 16, 16)]
            for j in range(64):
                vals = plsc.load_gather(gbuf.at[b], [rows, qv + j])
                stage[b, j, pl.ds(g * 16, 16)] = vals

    def unit(s, b):
        gather_dma(s, b).wait()
        @pl.when(s >= NBUF)
        def _():
            for tj in range(8):
                store_dma(s - NBUF, b, tj).wait()
        extract(s, b)
        for tj in range(8):
            store_dma(s, b, tj).start()
        @pl.when(s + NBUF < SEQ)
        def _():
            gather_dma(s + NBUF, b).start()

    def step(s3, _):
        for b in range(NBUF):
            unit(s3 * NBUF + b, b)
        return _

    lax.fori_loop(0, SEQ // NBUF, step, None)
    for b in range(NBUF):
        for tj in range(8):
            store_dma(SEQ - NBUF + b, b, tj).wait()


def kernel(target_item_batch, neg_item_batch, W2):
    target = target_item_batch.reshape(BATCH, TARGET_LEN)
    neg = neg_item_batch.reshape(BATCH, NEG_LEN)
    item_batch = jnp.concatenate([target, neg], axis=1)

    idx_t = item_batch.T.astype(jnp.int32)
    hidx = (idx_t >> 1).reshape(SEQ, NTB, 128)
    qoff = ((idx_t & 1) * NUM_FACTOR).reshape(SEQ, NTB, 128)

    tail_packed = W2[NBLK_FULL * IBLK:].reshape(32, 128)
    s1 = _pack_kernel(W2.T, tail_packed)
    out5 = _gather_kernel(s1, hidx, qoff)
    item_embedding = out5.transpose(2, 4, 0, 1, 3).reshape(
        BATCH, SEQ, NUM_FACTOR)
    return (item_batch, item_embedding)


# speedup vs baseline: 6.0842x; 6.0842x over previous
import functools

import jax
import jax.numpy as jnp
from jax import lax
from jax.experimental import pallas as pl
from jax.experimental.pallas import tpu as pltpu
from jax.experimental.pallas import tpu_sc as plsc

NUM_ITEM = 1000000
NUM_FACTOR = 64
BATCH = 4096
TARGET_LEN = 20
NEG_LEN = 100
SEQ = TARGET_LEN + NEG_LEN
TOTAL = BATCH * SEQ
HALF_ITEM = NUM_ITEM // 2

NC = 2
NS = 16
NW = NC * NS


CBLK = 8192
HSHIFT = 61 * CBLK
NTBLK = 62
NPACK = NTBLK * CBLK


def _table_tc_body(lo_ref, hi_ref, s1_ref):
    s1_ref[...] = jnp.concatenate([lo_ref[...].T, hi_ref[...].T], axis=1)


_table_tc = pl.pallas_call(
    _table_tc_body,
    grid=(NTBLK,),
    in_specs=[
        pl.BlockSpec((NUM_FACTOR, CBLK), lambda i: (0, i)),
        pl.BlockSpec((NUM_FACTOR, CBLK), lambda i: (0, i + 61)),
    ],
    out_specs=pl.BlockSpec((CBLK, 128), lambda i: (i, 0)),
    out_shape=jax.ShapeDtypeStruct((NPACK, 128), jnp.float32),
)


G = SEQ
ROWS_W = BATCH // NW
NBUF = 4
NSTEP = ROWS_W // NBUF

_mesh = plsc.VectorSubcoreMesh(
    core_axis_name="c", subcore_axis_name="s", num_cores=NC, num_subcores=NS)


@functools.partial(
    pl.kernel,
    out_type=jax.ShapeDtypeStruct((BATCH, SEQ, NUM_FACTOR), jnp.float32),
    mesh=_mesh,
    scratch_types=[
        pltpu.VMEM((ROWS_W, G), jnp.int32),
        pltpu.VMEM((NBUF, G, NUM_FACTOR), jnp.float32),
        pltpu.SemaphoreType.DMA((NBUF,)),
    ],
    compiler_params=pltpu.CompilerParams(use_tc_tiling_on_sc=False),
)
def _gather_sc(table_hbm, idx_hbm, out_hbm, idx_v, rows_v, gsems):
    wid = lax.axis_index("s") * NC + lax.axis_index("c")
    base = wid * ROWS_W

    pltpu.sync_copy(idx_hbm.at[wid], idx_v)

    def fire(g, b):
        pltpu.async_copy(table_hbm.at[idx_v.at[g]], rows_v.at[b], gsems.at[b])

    def wait_store(g, b):
        pltpu.make_async_copy(
            table_hbm.at[idx_v.at[g]], rows_v.at[b], gsems.at[b]).wait()
        pltpu.sync_copy(rows_v.at[b], out_hbm.at[base + g])

    for b in range(NBUF):
        fire(b, b)

    def outer(s, _):
        for b in range(NBUF):
            g = s * NBUF + b
            wait_store(g, b)
            fire(g + NBUF, b)
        return _

    lax.fori_loop(0, NSTEP - 1, outer, None)
    for b in range(NBUF):
        wait_store((NSTEP - 1) * NBUF + b, b)


def _out_tc_body(in_ref, out_ref):
    x3 = in_ref[...].reshape(128, 60, 128)
    for s2 in range(SEQ // 2):
        xt = x3[:, s2, :].T
        out_ref[2 * s2] = xt[:NUM_FACTOR].reshape(8, 1, 8, 128)
        out_ref[2 * s2 + 1] = xt[NUM_FACTOR:].reshape(8, 1, 8, 128)


_out_tc = pl.pallas_call(
    _out_tc_body,
    grid=(BATCH // 128,),
    in_specs=[pl.BlockSpec((SEQ // 2 * 128, 128), lambda i: (i, 0))],
    out_specs=pl.BlockSpec((SEQ, 8, 1, 8, 128), lambda i: (0, 0, i, 0, 0)),
    out_shape=jax.ShapeDtypeStruct((SEQ, 8, BATCH // 128, 8, 128),
                                   jnp.float32),
)


def kernel(target_item_batch, neg_item_batch, W2):
    target = target_item_batch.reshape(BATCH, TARGET_LEN)
    neg = neg_item_batch.reshape(BATCH, NEG_LEN)
    item_batch = jnp.concatenate([target, neg], axis=1)
    idx0 = item_batch.astype(jnp.int32)
    idx2 = jnp.where(idx0 < HSHIFT, 2 * idx0, 2 * (idx0 - HSHIFT) + 1)
    idx = idx2.reshape(NW, ROWS_W, G)

    wt = W2.T
    s1 = _table_tc(wt, wt)
    table = s1.reshape(2 * NPACK, NUM_FACTOR)
    emb_rm = _gather_sc(table, idx)
    out5 = _out_tc(emb_rm.reshape(TOTAL // 2, 128))
    item_embedding = out5.transpose(2, 4, 0, 1, 3).reshape(
        BATCH, SEQ, NUM_FACTOR)
    return (item_batch, item_embedding)


# speedup vs baseline: 6.2001x; 1.0190x over previous
import functools

import jax
import jax.numpy as jnp
from jax import lax
from jax.experimental import pallas as pl
from jax.experimental.pallas import tpu as pltpu
from jax.experimental.pallas import tpu_sc as plsc

NUM_ITEM = 1000000
NUM_FACTOR = 64
BATCH = 4096
TARGET_LEN = 20
NEG_LEN = 100
SEQ = TARGET_LEN + NEG_LEN
TOTAL = BATCH * SEQ
HALF_ITEM = NUM_ITEM // 2

NC = 2
NS = 16
NW = NC * NS


CBLK = 16384
HSHIFT = 30 * CBLK
NTBLK = 32
NPACK = NTBLK * CBLK


def _table_tc_body(lo_ref, hi_ref, s1_ref):
    s1_ref[:, :NUM_FACTOR] = lo_ref[...].T
    s1_ref[:, NUM_FACTOR:] = hi_ref[...].T


_table_tc = pl.pallas_call(
    _table_tc_body,
    grid=(NTBLK,),
    in_specs=[
        pl.BlockSpec((NUM_FACTOR, CBLK), lambda i: (0, i)),
        pl.BlockSpec((NUM_FACTOR, CBLK), lambda i: (0, i + 30)),
    ],
    out_specs=pl.BlockSpec((CBLK, 128), lambda i: (i, 0)),
    out_shape=jax.ShapeDtypeStruct((NPACK, 128), jnp.float32),
)


G = SEQ
ROWS_W = BATCH // NW
NBUF = 4
NSTEP = ROWS_W // NBUF

_mesh = plsc.VectorSubcoreMesh(
    core_axis_name="c", subcore_axis_name="s", num_cores=NC, num_subcores=NS)


@functools.partial(
    pl.kernel,
    out_type=jax.ShapeDtypeStruct((BATCH, SEQ, NUM_FACTOR), jnp.float32),
    mesh=_mesh,
    scratch_types=[
        pltpu.VMEM((ROWS_W, G), jnp.int32),
        pltpu.VMEM((NBUF, G, NUM_FACTOR), jnp.float32),
        pltpu.SemaphoreType.DMA((NBUF,)),
    ],
    compiler_params=pltpu.CompilerParams(use_tc_tiling_on_sc=False),
)
def _gather_sc(table_hbm, idx_hbm, out_hbm, idx_v, rows_v, gsems):
    wid = lax.axis_index("s") * NC + lax.axis_index("c")
    base = wid * ROWS_W

    pltpu.sync_copy(idx_hbm.at[wid], idx_v)

    def fire(g, b):
        pltpu.async_copy(table_hbm.at[idx_v.at[g]], rows_v.at[b], gsems.at[b])

    def wait_store(g, b):
        pltpu.make_async_copy(
            table_hbm.at[idx_v.at[g]], rows_v.at[b], gsems.at[b]).wait()
        pltpu.sync_copy(rows_v.at[b], out_hbm.at[base + g])

    for b in range(NBUF):
        fire(b, b)

    def outer(s, _):
        for b in range(NBUF):
            g = s * NBUF + b
            wait_store(g, b)
            fire(g + NBUF, b)
        return _

    lax.fori_loop(0, NSTEP - 1, outer, None)
    for b in range(NBUF):
        wait_store((NSTEP - 1) * NBUF + b, b)


def _out_tc_body(in_ref, out_ref):
    x3 = in_ref[...].reshape(128, 60, 128)
    for s2 in range(SEQ // 2):
        xt = x3[:, s2, :].T
        out_ref[2 * s2] = xt[:NUM_FACTOR].reshape(8, 1, 8, 128)
        out_ref[2 * s2 + 1] = xt[NUM_FACTOR:].reshape(8, 1, 8, 128)


_out_tc = pl.pallas_call(
    _out_tc_body,
    grid=(BATCH // 128,),
    in_specs=[pl.BlockSpec((SEQ // 2 * 128, 128), lambda i: (i, 0))],
    out_specs=pl.BlockSpec((SEQ, 8, 1, 8, 128), lambda i: (0, 0, i, 0, 0)),
    out_shape=jax.ShapeDtypeStruct((SEQ, 8, BATCH // 128, 8, 128),
                                   jnp.float32),
)


def kernel(target_item_batch, neg_item_batch, W2):
    target = target_item_batch.reshape(BATCH, TARGET_LEN)
    neg = neg_item_batch.reshape(BATCH, NEG_LEN)
    item_batch = jnp.concatenate([target, neg], axis=1)
    idx0 = item_batch.astype(jnp.int32)
    idx2 = jnp.where(idx0 < HSHIFT, 2 * idx0, 2 * (idx0 - HSHIFT) + 1)
    idx = idx2.reshape(NW, ROWS_W, G)

    wt = W2.T
    s1 = _table_tc(wt, wt)
    table = s1.reshape(2 * NPACK, NUM_FACTOR)
    emb_rm = _gather_sc(table, idx)
    out5 = _out_tc(emb_rm.reshape(TOTAL // 2, 128))
    item_embedding = out5.transpose(2, 4, 0, 1, 3).reshape(
        BATCH, SEQ, NUM_FACTOR)
    return (item_batch, item_embedding)
